# baseline (speedup 1.0000x reference)
import jax
import jax.numpy as jnp
from jax.experimental import pallas as pl
from jax.experimental.pallas import tpu as pltpu

_N = 16384
_C = 1000
_BR = 512
_NB = _N // _BR


def _dl_kernel(x_ref, t_ref, out_ref, acc_ref):
    i = pl.program_id(0)
    x = x_ref[...]
    col = jax.lax.broadcasted_iota(jnp.int32, x.shape, 1)
    mx = jnp.max(x, axis=1, keepdims=True)
    pred = jnp.min(jnp.where(x == mx, col, _C), axis=1, keepdims=True)
    tgt = t_ref[...]
    m = (pred != tgt).astype(jnp.float32)
    pf = pred.astype(jnp.float32)
    bn = jnp.sum(m)
    bs1 = jnp.sum(pf * m)
    bs2 = jnp.sum(pf * pf * m)

    @pl.when(i == 0)
    def _():
        acc_ref[0] = bn
        acc_ref[1] = bs1
        acc_ref[2] = bs2

    @pl.when(i != 0)
    def _():
        acc_ref[0] += bn
        acc_ref[1] += bs1
        acc_ref[2] += bs2

    @pl.when(i == _NB - 1)
    def _():
        n = acc_ref[0]
        s1 = acc_ref[1]
        s2 = acc_ref[2]
        mean = s1 / n
        var = (s2 - s1 * mean) / (n - 1.0)
        out_ref[0, 0] = 1.0 - jnp.sqrt(var)


def kernel(inputs, targets):
    tgt2 = targets.reshape(_N, 1)
    out = pl.pallas_call(
        _dl_kernel,
        grid=(_NB,),
        in_specs=[
            pl.BlockSpec((_BR, _C), lambda i: (i, 0)),
            pl.BlockSpec((_BR, 1), lambda i: (i, 0)),
        ],
        out_specs=pl.BlockSpec(
            (1, 1), lambda i: (0, 0), memory_space=pltpu.SMEM
        ),
        out_shape=jax.ShapeDtypeStruct((1, 1), jnp.float32),
        scratch_shapes=[pltpu.SMEM((3,), jnp.float32)],
        compiler_params=pltpu.CompilerParams(
            dimension_semantics=("arbitrary",),
        ),
    )(inputs, tgt2)
    return out.reshape(())


# speedup vs baseline: 1.1825x; 1.1825x over previous
import jax
import jax.numpy as jnp
from jax.experimental import pallas as pl
from jax.experimental.pallas import tpu as pltpu

_N = 16384
_C = 1000
_BR = 1024
_NB = _N // _BR
_TR = _BR // 128


def _dl_kernel(x_ref, t_ref, out_ref, acc_ref):
    i = pl.program_id(0)
    x = x_ref[...]
    col = jax.lax.broadcasted_iota(jnp.int32, x.shape, 1)
    mx = jnp.max(x, axis=1, keepdims=True)
    pred = jnp.min(jnp.where(x == mx, col, _C), axis=1, keepdims=True)
    pf = pred.astype(jnp.float32).reshape(_TR, 128, 1)
    r = jax.lax.broadcasted_iota(jnp.int32, (128, 128), 0)
    c = jax.lax.broadcasted_iota(jnp.int32, (128, 128), 1)
    eye = (r == c).astype(jnp.float32)
    pf_lane = jax.lax.dot_general(
        pf, eye, (((1,), (0,)), ((), ())),
        preferred_element_type=jnp.float32,
    ).reshape(_TR, 128)
    tf = t_ref[...].astype(jnp.float32)
    m = (pf_lane != tf).astype(jnp.float32)
    pm = pf_lane * m
    bn = jnp.sum(m)
    bs1 = jnp.sum(pm)
    bs2 = jnp.sum(pf_lane * pm)

    @pl.when(i == 0)
    def _():
        acc_ref[0] = bn
        acc_ref[1] = bs1
        acc_ref[2] = bs2

    @pl.when(i != 0)
    def _():
        acc_ref[0] += bn
        acc_ref[1] += bs1
        acc_ref[2] += bs2

    @pl.when(i == _NB - 1)
    def _():
        n = acc_ref[0]
        s1 = acc_ref[1]
        s2 = acc_ref[2]
        mean = s1 / n
        var = (s2 - s1 * mean) / (n - 1.0)
        out_ref[0, 0] = 1.0 - jnp.sqrt(var)


def kernel(inputs, targets):
    t128 = targets.reshape(128, 128)
    out = pl.pallas_call(
        _dl_kernel,
        grid=(_NB,),
        in_specs=[
            pl.BlockSpec((_BR, _C), lambda i: (i, 0)),
            pl.BlockSpec((_TR, 128), lambda i: (i, 0)),
        ],
        out_specs=pl.BlockSpec(
            (1, 1), lambda i: (0, 0), memory_space=pltpu.SMEM
        ),
        out_shape=jax.ShapeDtypeStruct((1, 1), jnp.float32),
        scratch_shapes=[pltpu.SMEM((3,), jnp.float32)],
        compiler_params=pltpu.CompilerParams(
            dimension_semantics=("arbitrary",),
        ),
    )(inputs, t128)
    return out.reshape(())


# speedup vs baseline: 3.5604x; 3.0110x over previous
import jax
import jax.numpy as jnp
from jax.experimental import pallas as pl
from jax.experimental.pallas import tpu as pltpu

_N = 16384
_C = 1000
_BN = 1024
_NB = _N // _BN
_TR = _BN // 128


def _dl_kernel(x_ref, t_ref, out_ref, acc_ref):
    i = pl.program_id(0)
    x = x_ref[...]
    parts = []
    for j in range(_TR):
        xc = x[:, j * 128:(j + 1) * 128]
        row = jax.lax.broadcasted_iota(jnp.int32, xc.shape, 0)
        mx = jnp.max(xc, axis=0, keepdims=True)
        parts.append(jnp.min(jnp.where(xc == mx, row, _C), axis=0, keepdims=True))
    pred = jnp.concatenate(parts, axis=0)
    tgt = t_ref[...]
    m = (pred != tgt).astype(jnp.float32)
    pf = pred.astype(jnp.float32)
    pm = pf * m
    bn = jnp.sum(m)
    bs1 = jnp.sum(pm)
    bs2 = jnp.sum(pf * pm)

    @pl.when(i == 0)
    def _():
        acc_ref[0] = bn
        acc_ref[1] = bs1
        acc_ref[2] = bs2

    @pl.when(i != 0)
    def _():
        acc_ref[0] += bn
        acc_ref[1] += bs1
        acc_ref[2] += bs2

    @pl.when(i == _NB - 1)
    def _():
        n = acc_ref[0]
        s1 = acc_ref[1]
        s2 = acc_ref[2]
        mean = s1 / n
        var = (s2 - s1 * mean) / (n - 1.0)
        out_ref[0, 0] = 1.0 - jnp.sqrt(var)


def kernel(inputs, targets):
    xt = inputs.T
    t128 = targets.reshape(128, 128)
    out = pl.pallas_call(
        _dl_kernel,
        grid=(_NB,),
        in_specs=[
            pl.BlockSpec((_C, _BN), lambda i: (0, i)),
            pl.BlockSpec((_TR, 128), lambda i: (i, 0)),
        ],
        out_specs=pl.BlockSpec(
            (1, 1), lambda i: (0, 0), memory_space=pltpu.SMEM
        ),
        out_shape=jax.ShapeDtypeStruct((1, 1), jnp.float32),
        scratch_shapes=[pltpu.SMEM((3,), jnp.float32)],
        compiler_params=pltpu.CompilerParams(
            dimension_semantics=("arbitrary",),
        ),
    )(xt, t128)
    return out.reshape(())


# speedup vs baseline: 4.1094x; 1.1542x over previous
import jax
import jax.numpy as jnp
from jax.experimental import pallas as pl
from jax.experimental.pallas import tpu as pltpu

_N = 16384
_C = 1000
_BN = 2048
_NB = _N // _BN
_TR = _BN // 128


def _dl_kernel(x_ref, t_ref, out_ref, acc_ref):
    i = pl.program_id(0)
    x = x_ref[...]
    parts = []
    for j in range(_TR):
        xc = x[:, j * 128:(j + 1) * 128]
        row = jax.lax.broadcasted_iota(jnp.int32, xc.shape, 0)
        mx = jnp.max(xc, axis=0, keepdims=True)
        parts.append(jnp.min(jnp.where(xc == mx, row, _C), axis=0, keepdims=True))
    pred = jnp.concatenate(parts, axis=0)
    tgt = t_ref[...]
    m = (pred != tgt).astype(jnp.float32)
    pf = pred.astype(jnp.float32)
    pm = pf * m
    bn = jnp.sum(m)
    bs1 = jnp.sum(pm)
    bs2 = jnp.sum(pf * pm)

    @pl.when(i == 0)
    def _():
        acc_ref[0] = bn
        acc_ref[1] = bs1
        acc_ref[2] = bs2

    @pl.when(i != 0)
    def _():
        acc_ref[0] += bn
        acc_ref[1] += bs1
        acc_ref[2] += bs2

    @pl.when(i == _NB - 1)
    def _():
        n = acc_ref[0]
        s1 = acc_ref[1]
        s2 = acc_ref[2]
        mean = s1 / n
        var = (s2 - s1 * mean) / (n - 1.0)
        out_ref[0, 0] = 1.0 - jnp.sqrt(var)


def kernel(inputs, targets):
    xt = inputs.T
    t128 = targets.reshape(128, 128)
    out = pl.pallas_call(
        _dl_kernel,
        grid=(_NB,),
        in_specs=[
            pl.BlockSpec((_C, _BN), lambda i: (0, i)),
            pl.BlockSpec((_TR, 128), lambda i: (i, 0)),
        ],
        out_specs=pl.BlockSpec(
            (1, 1), lambda i: (0, 0), memory_space=pltpu.SMEM
        ),
        out_shape=jax.ShapeDtypeStruct((1, 1), jnp.float32),
        scratch_shapes=[pltpu.SMEM((3,), jnp.float32)],
        compiler_params=pltpu.CompilerParams(
            dimension_semantics=("arbitrary",),
        ),
    )(xt, t128)
    return out.reshape(())
